# Initial kernel scaffold; baseline (speedup 1.0000x reference)
import jax
import jax.numpy as jnp
from jax.experimental import pallas as pl


def kernel(scores, relevance, eth_list):
    raise NotImplementedError("write your pallas kernel here")


# speedup vs baseline: 6.9257x; 6.9257x over previous
import math
import functools

import jax
import jax.numpy as jnp
from jax.experimental import pallas as pl

B = 1024
M = 200
G = 8
K = 10
LAM = 0.5
NUM_MC = 25

NB = 8
LOG_KFACT = math.log(float(math.factorial(K)))


def _body(scores_ref, rel_ref, eth_t_ref, g_ref, sig_ref, out_ref):
    step = pl.program_id(0)

    s = scores_ref[...]
    rel = rel_ref[...]
    gz = g_ref[...]
    sig = sig_ref[...]

    smax = jnp.max(s, axis=-1, keepdims=True)
    e = jnp.exp(s - smax)
    probs = e / jnp.sum(e, axis=-1, keepdims=True)
    T_p = jnp.sum(probs, axis=-1, keepdims=True)

    srel = jnp.sum(rel, axis=-1, keepdims=True)
    rel_n = jnp.where(srel > 0, rel / srel, 0.0)
    T_r = jnp.sum(rel_n, axis=-1, keepdims=True)

    z = s[:, None, :] + gz
    lane = jax.lax.broadcasted_iota(jnp.int32, (NB, NUM_MC, M), 2)

    probs_b = probs[:, None, :]
    rel_b = rel[:, None, :]
    rel_n_b = rel_n[:, None, :]

    q_list = []
    r_list = []
    code = jnp.zeros((NB, NUM_MC, M), jnp.int32)
    rn_top = jnp.zeros((NB, NUM_MC), jnp.float32)
    neg_inf = jnp.float32(-jnp.inf)
    for i in range(K):
        m = jnp.max(z, axis=-1, keepdims=True)
        eq = z == m
        isel = jnp.min(jnp.where(eq, lane, M), axis=-1, keepdims=True)
        onehot = lane == isel
        q_list.append(jnp.sum(jnp.where(onehot, probs_b, 0.0), axis=-1))
        r_list.append(jnp.sum(jnp.where(onehot, rel_b, 0.0), axis=-1))
        rn_top += jnp.sum(jnp.where(onehot, rel_n_b, 0.0), axis=-1)
        code = jnp.where(onehot, i + 1, code)
        z = jnp.where(onehot, neg_inf, z)

    sr = r_list[0]
    for i in range(1, K):
        sr = sr + r_list[i]
    inv_sr = jnp.where(sr > 0, 1.0 / sr, 0.0)
    w_list = [jnp.where(sr > 0, r_list[i] * inv_sr, 1.0 / K) for i in range(K)]

    W = jnp.zeros((NB, NUM_MC, M), jnp.float32)
    for i in range(K):
        W = jnp.where(code == i + 1, w_list[i][..., None], W)

    f_list = []
    for gdim in range(G):
        ethg = eth_t_ref[:, gdim, :]
        f_list.append(jnp.sum(W * ethg[:, None, :], axis=-1))
    SF = f_list[0]
    for gdim in range(1, G):
        SF = SF + f_list[gdim]
    inv_SF = 1.0 / SF
    H = jnp.zeros((NB, NUM_MC), jnp.float32)
    for gdim in range(G):
        p = f_list[gdim] * inv_SF
        H -= jnp.where(p > 0, p * jnp.log(jnp.where(p > 0, p, 1.0)), 0.0)

    logq = jnp.zeros((NB, NUM_MC), jnp.float32)
    logD = jnp.zeros((NB, NUM_MC), jnp.float32)
    acc = jnp.zeros((NB, NUM_MC), jnp.float32)
    for i in range(K):
        sig_i = sig[:, i * NUM_MC:(i + 1) * NUM_MC]
        qp_i = jnp.zeros((NB, NUM_MC), jnp.float32)
        for j in range(K):
            qp_i = jnp.where(sig_i == j, q_list[j], qp_i)
        D_i = T_p - acc
        logq += jnp.log(qp_i)
        logD += jnp.log(D_i)
        acc = acc + qp_i

    logprob = LOG_KFACT + logq - logD
    delta = 2.0 * rn_top - T_r
    reward = delta + LAM * H
    contrib = jnp.sum(logprob * reward) * (-1.0 / (NUM_MC * B))

    @pl.when(step == 0)
    def _():
        out_ref[...] = jnp.zeros_like(out_ref)
    out_ref[...] = out_ref[...] + contrib


def kernel(scores, relevance, eth_list, interpret=False):
    key = jax.random.key(1234)
    k_sample, k_perm = jax.random.split(key)
    g = jax.random.gumbel(k_sample, (B, NUM_MC, M), dtype=jnp.float32)
    sigma = jnp.argsort(jax.random.uniform(k_perm, (B, NUM_MC, K)), axis=-1)
    sig_r = sigma.transpose(0, 2, 1).reshape(B, K * NUM_MC).astype(jnp.int32)
    eth_t = eth_list.transpose(0, 2, 1)

    grid = (B // NB,)
    out = pl.pallas_call(
        _body,
        grid=grid,
        in_specs=[
            pl.BlockSpec((NB, M), lambda i: (i, 0)),
            pl.BlockSpec((NB, M), lambda i: (i, 0)),
            pl.BlockSpec((NB, G, M), lambda i: (i, 0, 0)),
            pl.BlockSpec((NB, NUM_MC, M), lambda i: (i, 0, 0)),
            pl.BlockSpec((NB, K * NUM_MC), lambda i: (i, 0)),
        ],
        out_specs=pl.BlockSpec((1, 1), lambda i: (0, 0)),
        out_shape=jax.ShapeDtypeStruct((1, 1), jnp.float32),
        interpret=interpret,
    )(scores, relevance, eth_t, g, sig_r)
    return out[0, 0]
